# baseline (speedup 1.0000x reference)
import functools

import jax
import jax.numpy as jnp
from jax import lax
from jax.experimental import pallas as pl
from jax.experimental.pallas import tpu as pltpu
from jax.experimental.pallas import tpu_sc as plsc

_N = 10000
_E = 160000
_G = 64
_IN_DIM = 11
_HID = 16
_E_DIM = 6
_EDGE_H = 64

try:
    _INFO = plsc.get_sparse_core_info()
    _NC = _INFO.num_cores
    _NS = _INFO.num_subcores
except ValueError:
    _NC, _NS = 2, 16
_NW = _NC * _NS
_CH = 128
_N_CH = 40
_PER_W = _N_CH * _CH
_E_PAD = _NW * _PER_W
_ROWS_S = 640
_N_AGG = _NS * _ROWS_S
_GRP = 8

@functools.lru_cache
def _get_mesh():
    return plsc.VectorSubcoreMesh(
        core_axis_name="c", subcore_axis_name="s", num_cores=_NC, num_subcores=_NS
    )


@functools.lru_cache
def _sc_gather_fn():
    @functools.partial(
        pl.kernel,
        out_type=jax.ShapeDtypeStruct((_E_PAD, _HID), jnp.float32),
        mesh=_get_mesh(),
        scratch_types=[
            pltpu.VMEM((_N_CH, _CH), jnp.int32),
            pltpu.VMEM((_PER_W, _HID), jnp.float32),
            pltpu.SemaphoreType.DMA,
        ],
        compiler_params=pltpu.CompilerParams(
            use_tc_tiling_on_sc=False, skip_device_barrier=True
        ),
    )
    def _sc_gather(table_hbm, idx_hbm, out_hbm, idx_v, rows_v, sem):
        wid = lax.axis_index("s") * _NC + lax.axis_index("c")
        pltpu.sync_copy(idx_hbm.at[wid], idx_v)

        def body(g, carry):
            base = g * _GRP
            descs = [
                pltpu.async_copy(
                    table_hbm.at[idx_v.at[base + t]],
                    rows_v.at[pl.ds((base + t) * _CH, _CH)],
                    sem,
                )
                for t in range(_GRP)
            ]
            for d in descs:
                d.wait()
            return carry

        lax.fori_loop(0, _N_CH // _GRP, body, 0)
        pltpu.sync_copy(rows_v, out_hbm.at[pl.ds(wid * _PER_W, _PER_W)])

    return _sc_gather


@functools.lru_cache
def _make_sc_scatter(with_deg):
    n_out = 2 if with_deg else 1
    out_type = [jax.ShapeDtypeStruct((_NC, _N_AGG, _HID), jnp.float32)] * n_out
    scratch = [
        pltpu.VMEM((_N_CH, _CH), jnp.int32),
        pltpu.VMEM((_PER_W, _HID), jnp.float32),
        pltpu.VMEM((_CH, _HID), jnp.float32),
        pltpu.VMEM_SHARED((_N_AGG, _HID), jnp.float32),
        pltpu.SemaphoreType.DMA,
    ]
    if with_deg:
        scratch.append(pltpu.VMEM_SHARED((_N_AGG, _HID), jnp.float32))

    @functools.partial(
        pl.kernel, out_type=out_type, mesh=_get_mesh(), scratch_types=scratch,
        compiler_params=pltpu.CompilerParams(
            use_tc_tiling_on_sc=False, skip_device_barrier=True
        ),
    )
    def _sc_scatter(msg_hbm, dst_hbm, zeros_hbm, ones_hbm, *rest):
        if with_deg:
            agg_out, deg_out, idx_v, rows_v, ones_v, agg_sh, sem, deg_sh = rest
        else:
            agg_out, idx_v, rows_v, ones_v, agg_sh, sem = rest
        cid = lax.axis_index("c")
        sid = lax.axis_index("s")
        wid = sid * _NC + cid
        r0 = sid * _ROWS_S
        pltpu.sync_copy(zeros_hbm.at[pl.ds(r0, _ROWS_S)], agg_sh.at[pl.ds(r0, _ROWS_S)])
        if with_deg:
            pltpu.sync_copy(zeros_hbm.at[pl.ds(r0, _ROWS_S)], deg_sh.at[pl.ds(r0, _ROWS_S)])
            pltpu.sync_copy(ones_hbm, ones_v)
        pltpu.sync_copy(dst_hbm.at[wid], idx_v)
        pltpu.sync_copy(msg_hbm.at[pl.ds(wid * _PER_W, _PER_W)], rows_v)
        plsc.subcore_barrier()

        def body(j, carry):
            pltpu.sync_copy(
                rows_v.at[pl.ds(j * _CH, _CH)], agg_sh.at[idx_v.at[j]], add=True
            )
            if with_deg:
                pltpu.sync_copy(ones_v, deg_sh.at[idx_v.at[j]], add=True)
            return carry

        lax.fori_loop(0, _N_CH, body, 0)
        plsc.subcore_barrier()
        pltpu.sync_copy(agg_sh.at[pl.ds(r0, _ROWS_S)], agg_out.at[cid, pl.ds(r0, _ROWS_S)])
        if with_deg:
            pltpu.sync_copy(deg_sh.at[pl.ds(r0, _ROWS_S)], deg_out.at[cid, pl.ds(r0, _ROWS_S)])

    return _sc_scatter


_EB = 4096


def _tc_edge(ea_t, xs_t, W1, b1, W2, b2, din):
    def body(ea_ref, xs_ref, w1t_ref, b1_ref, w2t_ref, b2_ref, out_ref):
        eh_t = jnp.maximum(
            jnp.dot(w1t_ref[...], ea_ref[...], preferred_element_type=jnp.float32)
            + b1_ref[...],
            0.0,
        )
        p_t = (
            jnp.dot(w2t_ref[...], eh_t, preferred_element_type=jnp.float32)
            + b2_ref[...]
        )
        xs_t_b = xs_ref[...]
        acc = xs_t_b[0:1, :] * p_t[0:_HID, :]
        for i in range(1, din):
            acc = acc + xs_t_b[i : i + 1, :] * p_t[i * _HID : (i + 1) * _HID, :]
        out_ref[...] = acc

    grid = (_E_PAD // _EB,)
    return pl.pallas_call(
        body,
        grid=grid,
        in_specs=[
            pl.BlockSpec((_E_DIM, _EB), lambda i: (0, i)),
            pl.BlockSpec((_HID, _EB), lambda i: (0, i)),
            pl.BlockSpec((_EDGE_H, _E_DIM), lambda i: (0, 0)),
            pl.BlockSpec((_EDGE_H, 1), lambda i: (0, 0)),
            pl.BlockSpec((din * _HID, _EDGE_H), lambda i: (0, 0)),
            pl.BlockSpec((din * _HID, 1), lambda i: (0, 0)),
        ],
        out_specs=pl.BlockSpec((_HID, _EB), lambda i: (0, i)),
        out_shape=jax.ShapeDtypeStruct((_HID, _E_PAD), jnp.float32),
    )(ea_t, xs_t, W1.T, b1[:, None], W2.T, b2[:, None])


def _tc_norm(agg2, deg2, bias, gamma, beta):
    def body(a_ref, d_ref, bias_ref, g_ref, be_ref, out_ref):
        a = a_ref[0] + a_ref[1]
        d = d_ref[0] + d_ref[1]
        t = a / jnp.maximum(d, 1.0) + bias_ref[...]
        r = jnp.maximum(t, 0.0)
        m = jnp.mean(r, axis=0, keepdims=True)
        v = jnp.mean((r - m) * (r - m), axis=0, keepdims=True)
        out_ref[...] = (r - m) / jnp.sqrt(v + 1e-5) * g_ref[...] + be_ref[...]

    return pl.pallas_call(
        body,
        out_shape=jax.ShapeDtypeStruct((_N, _HID), jnp.float32),
    )(agg2, deg2, bias[None], gamma[None], beta[None])


def _tc_readout(h, gid, gamma, beta):
    def body(h_ref, gid_ref, g_ref, be_ref, out_ref, hx_ref):
        hv = h_ref[...]
        gids = gid_ref[...]
        iota_g = lax.broadcasted_iota(jnp.int32, (_N, _G), 1)
        maskf = (gids == iota_g).astype(jnp.float32)
        counts = jnp.sum(maskf, axis=0)[:, None]
        sums = lax.dot_general(
            maskf, hv, (((0,), (0,)), ((), ())),
            preferred_element_type=jnp.float32,
        )
        hn = sums / jnp.maximum(counts, 1.0)
        m = jnp.mean(hn, axis=0, keepdims=True)
        v = jnp.mean((hn - m) * (hn - m), axis=0, keepdims=True)
        hnb = (hn - m) / jnp.sqrt(v + 1e-5) * g_ref[...] + be_ref[...]

        def body_g(g, carry):
            mk = gids == g
            mx = jnp.max(jnp.where(mk, hv, -jnp.inf), axis=0)
            hx_ref[pl.ds(g, 1), :] = mx[None, :]
            return carry

        lax.fori_loop(0, _G, body_g, 0)
        out_ref[:, 0:_HID] = hnb
        out_ref[:, _HID : 2 * _HID] = hx_ref[...]

    return pl.pallas_call(
        body,
        out_shape=jax.ShapeDtypeStruct((_G, 2 * _HID), jnp.float32),
        scratch_shapes=[pltpu.VMEM((_G, _HID), jnp.float32)],
    )(h, gid[:, None], gamma[None], beta[None])


def kernel(x, edge_index, edge_attr, node_graph_ids, params):
    src = edge_index[0]
    dst = edge_index[1]
    pad_e = _E_PAD - _E
    src3 = jnp.concatenate([src, jnp.zeros((pad_e,), jnp.int32)]).reshape(
        _NW, _N_CH, _CH
    )
    dst3 = jnp.concatenate([dst, jnp.full((pad_e,), _N, jnp.int32)]).reshape(
        _NW, _N_CH, _CH
    )
    ea_t = jnp.concatenate(
        [edge_attr, jnp.zeros((pad_e, _E_DIM), jnp.float32)], axis=0
    ).T
    x_pad = jnp.concatenate(
        [x, jnp.zeros((_N, _HID - _IN_DIM), jnp.float32)], axis=1
    )
    zeros_init = jnp.zeros((_N_AGG, _HID), jnp.float32)
    ones_rows = jnp.ones((_CH, _HID), jnp.float32)

    layers = params["layers"]
    h = x_pad
    deg2 = None
    for li, din in enumerate((_IN_DIM, _HID)):
        lp = layers[li]
        xs = _sc_gather_fn()(h, src3)
        msg_t = _tc_edge(ea_t, xs.T, lp["W1"], lp["b1"], lp["W2"], lp["b2"], din)
        msg = msg_t.T
        if li == 0:
            agg2, deg2 = _make_sc_scatter(True)(msg, dst3, zeros_init, ones_rows)
        else:
            agg2 = _make_sc_scatter(False)(msg, dst3, zeros_init, ones_rows)
            if isinstance(agg2, (list, tuple)):
                agg2 = agg2[0]
        h = _tc_norm(
            agg2[:, :_N], deg2[:, :_N], lp["bias"], lp["gamma"], lp["beta"]
        )

    return _tc_readout(
        h, node_graph_ids, params["bn_out_gamma"], params["bn_out_beta"]
    )


# speedup vs baseline: 1.0970x; 1.0970x over previous
import functools

import jax
import jax.numpy as jnp
from jax import lax
from jax.experimental import pallas as pl
from jax.experimental.pallas import tpu as pltpu
from jax.experimental.pallas import tpu_sc as plsc

_N = 10000
_E = 160000
_G = 64
_IN_DIM = 11
_HID = 16
_E_DIM = 6
_EDGE_H = 64

try:
    _INFO = plsc.get_sparse_core_info()
    _NC = _INFO.num_cores
    _NS = _INFO.num_subcores
except ValueError:
    _NC, _NS = 2, 16
_NW = _NC * _NS
_CH = 128
_N_CH = 40
_PER_W = _N_CH * _CH
_E_PAD = _NW * _PER_W
_ROWS_S = 640
_N_AGG = _NS * _ROWS_S
_GRP = 8

@functools.lru_cache
def _get_mesh():
    return plsc.VectorSubcoreMesh(
        core_axis_name="c", subcore_axis_name="s", num_cores=_NC, num_subcores=_NS
    )


@functools.lru_cache
def _sc_gather_fn():
    @functools.partial(
        pl.kernel,
        out_type=jax.ShapeDtypeStruct((_E_PAD, _HID), jnp.float32),
        mesh=_get_mesh(),
        scratch_types=[
            pltpu.VMEM((_N_CH, _CH), jnp.int32),
            pltpu.VMEM((_PER_W, _HID), jnp.float32),
            pltpu.SemaphoreType.DMA,
        ],
        compiler_params=pltpu.CompilerParams(
            use_tc_tiling_on_sc=False, skip_device_barrier=True
        ),
    )
    def _sc_gather(table_hbm, idx_hbm, out_hbm, idx_v, rows_v, sem):
        wid = lax.axis_index("s") * _NC + lax.axis_index("c")
        pltpu.sync_copy(idx_hbm.at[wid], idx_v)

        def body(g, carry):
            base = g * _GRP
            descs = [
                pltpu.async_copy(
                    table_hbm.at[idx_v.at[base + t]],
                    rows_v.at[pl.ds((base + t) * _CH, _CH)],
                    sem,
                )
                for t in range(_GRP)
            ]
            for d in descs:
                d.wait()
            return carry

        lax.fori_loop(0, _N_CH // _GRP, body, 0)
        pltpu.sync_copy(rows_v, out_hbm.at[pl.ds(wid * _PER_W, _PER_W)])

    return _sc_gather


@functools.lru_cache
def _make_sc_scatter(with_deg):
    n_out = 2 if with_deg else 1
    out_type = [jax.ShapeDtypeStruct((_NC, _N_AGG, _HID), jnp.float32)] * n_out
    scratch = [
        pltpu.VMEM((_N_CH, _CH), jnp.int32),
        pltpu.VMEM((_PER_W, _HID), jnp.float32),
        pltpu.VMEM((_CH, _HID), jnp.float32),
        pltpu.VMEM_SHARED((_N_AGG, _HID), jnp.float32),
        pltpu.SemaphoreType.DMA,
    ]
    if with_deg:
        scratch.append(pltpu.VMEM_SHARED((_N_AGG, _HID), jnp.float32))

    @functools.partial(
        pl.kernel, out_type=out_type, mesh=_get_mesh(), scratch_types=scratch,
        compiler_params=pltpu.CompilerParams(
            use_tc_tiling_on_sc=False, skip_device_barrier=True
        ),
    )
    def _sc_scatter(msg_hbm, dst_hbm, zeros_hbm, ones_hbm, *rest):
        if with_deg:
            agg_out, deg_out, idx_v, rows_v, ones_v, agg_sh, sem, deg_sh = rest
        else:
            agg_out, idx_v, rows_v, ones_v, agg_sh, sem = rest
        cid = lax.axis_index("c")
        sid = lax.axis_index("s")
        wid = sid * _NC + cid
        r0 = sid * _ROWS_S
        pltpu.sync_copy(zeros_hbm.at[pl.ds(r0, _ROWS_S)], agg_sh.at[pl.ds(r0, _ROWS_S)])
        if with_deg:
            pltpu.sync_copy(zeros_hbm.at[pl.ds(r0, _ROWS_S)], deg_sh.at[pl.ds(r0, _ROWS_S)])
            pltpu.sync_copy(ones_hbm, ones_v)
        pltpu.sync_copy(dst_hbm.at[wid], idx_v)
        pltpu.sync_copy(msg_hbm.at[pl.ds(wid * _PER_W, _PER_W)], rows_v)
        plsc.subcore_barrier()

        def body(j, carry):
            pltpu.sync_copy(
                rows_v.at[pl.ds(j * _CH, _CH)], agg_sh.at[idx_v.at[j]], add=True
            )
            if with_deg:
                pltpu.sync_copy(ones_v, deg_sh.at[idx_v.at[j]], add=True)
            return carry

        lax.fori_loop(0, _N_CH, body, 0)
        plsc.subcore_barrier()
        pltpu.sync_copy(agg_sh.at[pl.ds(r0, _ROWS_S)], agg_out.at[cid, pl.ds(r0, _ROWS_S)])
        if with_deg:
            pltpu.sync_copy(deg_sh.at[pl.ds(r0, _ROWS_S)], deg_out.at[cid, pl.ds(r0, _ROWS_S)])

    return _sc_scatter


_EB = 4096


def _tc_edge(ea_t, xs_t, W1, b1, W2, b2, din):
    def body(ea_ref, xs_ref, w1t_ref, b1_ref, w2t_ref, b2_ref, out_ref):
        eh_t = jnp.maximum(
            jnp.dot(w1t_ref[...], ea_ref[...], preferred_element_type=jnp.float32)
            + b1_ref[...],
            0.0,
        )
        p_t = (
            jnp.dot(w2t_ref[...], eh_t, preferred_element_type=jnp.float32)
            + b2_ref[...]
        )
        xs_t_b = xs_ref[...].T
        acc = xs_t_b[0:1, :] * p_t[0:_HID, :]
        for i in range(1, din):
            acc = acc + xs_t_b[i : i + 1, :] * p_t[i * _HID : (i + 1) * _HID, :]
        out_ref[...] = acc.T

    grid = (_E_PAD // _EB,)
    return pl.pallas_call(
        body,
        grid=grid,
        in_specs=[
            pl.BlockSpec((_E_DIM, _EB), lambda i: (0, i)),
            pl.BlockSpec((_EB, _HID), lambda i: (i, 0)),
            pl.BlockSpec((_EDGE_H, _E_DIM), lambda i: (0, 0)),
            pl.BlockSpec((_EDGE_H, 1), lambda i: (0, 0)),
            pl.BlockSpec((din * _HID, _EDGE_H), lambda i: (0, 0)),
            pl.BlockSpec((din * _HID, 1), lambda i: (0, 0)),
        ],
        out_specs=pl.BlockSpec((_EB, _HID), lambda i: (i, 0)),
        out_shape=jax.ShapeDtypeStruct((_E_PAD, _HID), jnp.float32),
    )(ea_t, xs_t, W1.T, b1[:, None], W2.T, b2[:, None])


def _tc_norm(agg2, deg2, bias, gamma, beta):
    def body(a_ref, d_ref, bias_ref, g_ref, be_ref, out_ref):
        a = a_ref[0] + a_ref[1]
        d = d_ref[0] + d_ref[1]
        t = a / jnp.maximum(d, 1.0) + bias_ref[...]
        r = jnp.maximum(t, 0.0)
        m = jnp.mean(r, axis=0, keepdims=True)
        v = jnp.mean((r - m) * (r - m), axis=0, keepdims=True)
        out_ref[...] = (r - m) / jnp.sqrt(v + 1e-5) * g_ref[...] + be_ref[...]

    return pl.pallas_call(
        body,
        out_shape=jax.ShapeDtypeStruct((_N, _HID), jnp.float32),
    )(agg2, deg2, bias[None], gamma[None], beta[None])


def _tc_readout(h, gid, gamma, beta):
    def body(h_ref, gid_ref, g_ref, be_ref, out_ref, hx_ref):
        hv = h_ref[...]
        gids = gid_ref[...]
        iota_g = lax.broadcasted_iota(jnp.int32, (_N, _G), 1)
        maskf = (gids == iota_g).astype(jnp.float32)
        counts = jnp.sum(maskf, axis=0)[:, None]
        sums = lax.dot_general(
            maskf, hv, (((0,), (0,)), ((), ())),
            preferred_element_type=jnp.float32,
        )
        hn = sums / jnp.maximum(counts, 1.0)
        m = jnp.mean(hn, axis=0, keepdims=True)
        v = jnp.mean((hn - m) * (hn - m), axis=0, keepdims=True)
        hnb = (hn - m) / jnp.sqrt(v + 1e-5) * g_ref[...] + be_ref[...]

        def body_g(g, carry):
            mk = gids == g
            mx = jnp.max(jnp.where(mk, hv, -jnp.inf), axis=0)
            hx_ref[pl.ds(g, 1), :] = mx[None, :]
            return carry

        lax.fori_loop(0, _G, body_g, 0)
        out_ref[:, 0:_HID] = hnb
        out_ref[:, _HID : 2 * _HID] = hx_ref[...]

    return pl.pallas_call(
        body,
        out_shape=jax.ShapeDtypeStruct((_G, 2 * _HID), jnp.float32),
        scratch_shapes=[pltpu.VMEM((_G, _HID), jnp.float32)],
    )(h, gid[:, None], gamma[None], beta[None])


def kernel(x, edge_index, edge_attr, node_graph_ids, params):
    src = edge_index[0]
    dst = edge_index[1]
    pad_e = _E_PAD - _E
    src3 = jnp.concatenate([src, jnp.zeros((pad_e,), jnp.int32)]).reshape(
        _NW, _N_CH, _CH
    )
    dst3 = jnp.concatenate([dst, jnp.full((pad_e,), _N, jnp.int32)]).reshape(
        _NW, _N_CH, _CH
    )
    ea_t = jnp.concatenate(
        [edge_attr, jnp.zeros((pad_e, _E_DIM), jnp.float32)], axis=0
    ).T
    x_pad = jnp.concatenate(
        [x, jnp.zeros((_N, _HID - _IN_DIM), jnp.float32)], axis=1
    )
    zeros_init = jnp.zeros((_N_AGG, _HID), jnp.float32)
    ones_rows = jnp.ones((_CH, _HID), jnp.float32)

    layers = params["layers"]
    h = x_pad
    deg2 = None
    for li, din in enumerate((_IN_DIM, _HID)):
        lp = layers[li]
        xs = _sc_gather_fn()(h, src3)
        msg = _tc_edge(ea_t, xs, lp["W1"], lp["b1"], lp["W2"], lp["b2"], din)
        if li == 0:
            agg2, deg2 = _make_sc_scatter(True)(msg, dst3, zeros_init, ones_rows)
        else:
            agg2 = _make_sc_scatter(False)(msg, dst3, zeros_init, ones_rows)
            if isinstance(agg2, (list, tuple)):
                agg2 = agg2[0]
        h = _tc_norm(
            agg2[:, :_N], deg2[:, :_N], lp["bias"], lp["gamma"], lp["beta"]
        )

    return _tc_readout(
        h, node_graph_ids, params["bn_out_gamma"], params["bn_out_beta"]
    )


# speedup vs baseline: 1.1054x; 1.0077x over previous
import functools

import jax
import jax.numpy as jnp
from jax import lax
from jax.experimental import pallas as pl
from jax.experimental.pallas import tpu as pltpu
from jax.experimental.pallas import tpu_sc as plsc

_N = 10000
_E = 160000
_G = 64
_IN_DIM = 11
_HID = 16
_E_DIM = 6
_EDGE_H = 64

try:
    _INFO = plsc.get_sparse_core_info()
    _NC = _INFO.num_cores
    _NS = _INFO.num_subcores
except ValueError:
    _NC, _NS = 2, 16
_NW = _NC * _NS
_CH = 128
_N_CH = 40
_PER_W = _N_CH * _CH
_E_PAD = _NW * _PER_W
_ROWS_S = 640
_N_AGG = _NS * _ROWS_S
_GRP = 8

@functools.lru_cache
def _get_mesh():
    return plsc.VectorSubcoreMesh(
        core_axis_name="c", subcore_axis_name="s", num_cores=_NC, num_subcores=_NS
    )


@functools.lru_cache
def _sc_gather_fn():
    @functools.partial(
        pl.kernel,
        out_type=jax.ShapeDtypeStruct((_E_PAD, _HID), jnp.float32),
        mesh=_get_mesh(),
        scratch_types=[
            pltpu.VMEM((_N_CH, _CH), jnp.int32),
            pltpu.VMEM((_PER_W, _HID), jnp.float32),
            pltpu.SemaphoreType.DMA,
        ],
        compiler_params=pltpu.CompilerParams(
            use_tc_tiling_on_sc=False, skip_device_barrier=True
        ),
    )
    def _sc_gather(table_hbm, idx_hbm, out_hbm, idx_v, rows_v, sem):
        wid = lax.axis_index("s") * _NC + lax.axis_index("c")
        pltpu.sync_copy(idx_hbm.at[wid], idx_v)

        def body(g, carry):
            base = g * _GRP
            descs = [
                pltpu.async_copy(
                    table_hbm.at[idx_v.at[base + t]],
                    rows_v.at[pl.ds((base + t) * _CH, _CH)],
                    sem,
                )
                for t in range(_GRP)
            ]
            for d in descs:
                d.wait()
            return carry

        lax.fori_loop(0, _N_CH // _GRP, body, 0)
        pltpu.sync_copy(rows_v, out_hbm.at[pl.ds(wid * _PER_W, _PER_W)])

    return _sc_gather


@functools.lru_cache
def _make_sc_scatter(with_deg):
    n_out = 2 if with_deg else 1
    out_type = [jax.ShapeDtypeStruct((_NC, _N_AGG, _HID), jnp.float32)] * n_out
    scratch = [
        pltpu.VMEM((_N_CH, _CH), jnp.int32),
        pltpu.VMEM((_PER_W, _HID), jnp.float32),
        pltpu.VMEM((_CH, _HID), jnp.float32),
        pltpu.VMEM_SHARED((_N_AGG, _HID), jnp.float32),
        pltpu.SemaphoreType.DMA,
    ]
    if with_deg:
        scratch.append(pltpu.VMEM_SHARED((_N_AGG, _HID), jnp.float32))

    @functools.partial(
        pl.kernel, out_type=out_type, mesh=_get_mesh(), scratch_types=scratch,
        compiler_params=pltpu.CompilerParams(
            use_tc_tiling_on_sc=False, skip_device_barrier=True
        ),
    )
    def _sc_scatter(msg_hbm, dst_hbm, zeros_hbm, ones_hbm, *rest):
        if with_deg:
            agg_out, deg_out, idx_v, rows_v, ones_v, agg_sh, sem, deg_sh = rest
        else:
            agg_out, idx_v, rows_v, ones_v, agg_sh, sem = rest
        cid = lax.axis_index("c")
        sid = lax.axis_index("s")
        wid = sid * _NC + cid
        r0 = sid * _ROWS_S
        pltpu.sync_copy(zeros_hbm.at[pl.ds(r0, _ROWS_S)], agg_sh.at[pl.ds(r0, _ROWS_S)])
        if with_deg:
            pltpu.sync_copy(zeros_hbm.at[pl.ds(r0, _ROWS_S)], deg_sh.at[pl.ds(r0, _ROWS_S)])
            pltpu.sync_copy(ones_hbm, ones_v)
        pltpu.sync_copy(dst_hbm.at[wid], idx_v)
        pltpu.sync_copy(msg_hbm.at[pl.ds(wid * _PER_W, _PER_W)], rows_v)
        plsc.subcore_barrier()

        def body(j, carry):
            d1 = pltpu.async_copy(
                rows_v.at[pl.ds(j * _CH, _CH)], agg_sh.at[idx_v.at[j]], sem,
                add=True,
            )
            if with_deg:
                d2 = pltpu.async_copy(
                    ones_v, deg_sh.at[idx_v.at[j]], sem, add=True
                )
            d1.wait()
            if with_deg:
                d2.wait()
            return carry

        lax.fori_loop(0, _N_CH, body, 0)
        plsc.subcore_barrier()
        pltpu.sync_copy(agg_sh.at[pl.ds(r0, _ROWS_S)], agg_out.at[cid, pl.ds(r0, _ROWS_S)])
        if with_deg:
            pltpu.sync_copy(deg_sh.at[pl.ds(r0, _ROWS_S)], deg_out.at[cid, pl.ds(r0, _ROWS_S)])

    return _sc_scatter


_EB = 4096


def _tc_edge(ea_t, xs_t, W1, b1, W2, b2, din):
    def body(ea_ref, xs_ref, w1t_ref, b1_ref, w2t_ref, b2_ref, out_ref):
        eh_t = jnp.maximum(
            jnp.dot(w1t_ref[...], ea_ref[...], preferred_element_type=jnp.float32)
            + b1_ref[...],
            0.0,
        )
        p_t = (
            jnp.dot(w2t_ref[...], eh_t, preferred_element_type=jnp.float32)
            + b2_ref[...]
        )
        xs_t_b = xs_ref[...].T
        acc = xs_t_b[0:1, :] * p_t[0:_HID, :]
        for i in range(1, din):
            acc = acc + xs_t_b[i : i + 1, :] * p_t[i * _HID : (i + 1) * _HID, :]
        out_ref[...] = acc.T

    grid = (_E_PAD // _EB,)
    return pl.pallas_call(
        body,
        grid=grid,
        in_specs=[
            pl.BlockSpec((_E_DIM, _EB), lambda i: (0, i)),
            pl.BlockSpec((_EB, _HID), lambda i: (i, 0)),
            pl.BlockSpec((_EDGE_H, _E_DIM), lambda i: (0, 0)),
            pl.BlockSpec((_EDGE_H, 1), lambda i: (0, 0)),
            pl.BlockSpec((din * _HID, _EDGE_H), lambda i: (0, 0)),
            pl.BlockSpec((din * _HID, 1), lambda i: (0, 0)),
        ],
        out_specs=pl.BlockSpec((_EB, _HID), lambda i: (i, 0)),
        out_shape=jax.ShapeDtypeStruct((_E_PAD, _HID), jnp.float32),
    )(ea_t, xs_t, W1.T, b1[:, None], W2.T, b2[:, None])


def _tc_norm(agg2, deg2, bias, gamma, beta):
    def body(a_ref, d_ref, bias_ref, g_ref, be_ref, out_ref):
        a = a_ref[0] + a_ref[1]
        d = d_ref[0] + d_ref[1]
        t = a / jnp.maximum(d, 1.0) + bias_ref[...]
        r = jnp.maximum(t, 0.0)
        m = jnp.mean(r, axis=0, keepdims=True)
        v = jnp.mean((r - m) * (r - m), axis=0, keepdims=True)
        out_ref[...] = (r - m) / jnp.sqrt(v + 1e-5) * g_ref[...] + be_ref[...]

    return pl.pallas_call(
        body,
        out_shape=jax.ShapeDtypeStruct((_N, _HID), jnp.float32),
    )(agg2, deg2, bias[None], gamma[None], beta[None])


def _tc_final(agg2, deg2, bias, gamma, beta, gid, g2, be2):
    def body(a_ref, d_ref, bias_ref, lg_ref, lbe_ref, gid_ref, g_ref, be_ref,
             out_ref, hx_ref):
        a = a_ref[0] + a_ref[1]
        d = d_ref[0] + d_ref[1]
        t = a / jnp.maximum(d, 1.0) + bias_ref[...]
        r = jnp.maximum(t, 0.0)
        lm = jnp.mean(r, axis=0, keepdims=True)
        lv = jnp.mean((r - lm) * (r - lm), axis=0, keepdims=True)
        hv = (r - lm) / jnp.sqrt(lv + 1e-5) * lg_ref[...] + lbe_ref[...]
        gids = gid_ref[...]
        iota_g = lax.broadcasted_iota(jnp.int32, (_N, _G), 1)
        maskf = (gids == iota_g).astype(jnp.float32)
        counts = jnp.sum(maskf, axis=0)[:, None]
        sums = lax.dot_general(
            maskf, hv, (((0,), (0,)), ((), ())),
            preferred_element_type=jnp.float32,
        )
        hn = sums / jnp.maximum(counts, 1.0)
        m = jnp.mean(hn, axis=0, keepdims=True)
        v = jnp.mean((hn - m) * (hn - m), axis=0, keepdims=True)
        hnb = (hn - m) / jnp.sqrt(v + 1e-5) * g_ref[...] + be_ref[...]

        def body_g(g, carry):
            mk = gids == g
            mx = jnp.max(jnp.where(mk, hv, -jnp.inf), axis=0)
            hx_ref[pl.ds(g, 1), :] = mx[None, :]
            return carry

        lax.fori_loop(0, _G, body_g, 0)
        out_ref[:, 0:_HID] = hnb
        out_ref[:, _HID : 2 * _HID] = hx_ref[...]

    return pl.pallas_call(
        body,
        out_shape=jax.ShapeDtypeStruct((_G, 2 * _HID), jnp.float32),
        scratch_shapes=[pltpu.VMEM((_G, _HID), jnp.float32)],
    )(agg2, deg2, bias[None], gamma[None], beta[None], gid[:, None],
      g2[None], be2[None])


def kernel(x, edge_index, edge_attr, node_graph_ids, params):
    src = edge_index[0]
    dst = edge_index[1]
    pad_e = _E_PAD - _E
    src3 = jnp.concatenate([src, jnp.zeros((pad_e,), jnp.int32)]).reshape(
        _NW, _N_CH, _CH
    )
    dst3 = jnp.concatenate([dst, jnp.full((pad_e,), _N, jnp.int32)]).reshape(
        _NW, _N_CH, _CH
    )
    ea_t = jnp.concatenate(
        [edge_attr, jnp.zeros((pad_e, _E_DIM), jnp.float32)], axis=0
    ).T
    x_pad = jnp.concatenate(
        [x, jnp.zeros((_N, _HID - _IN_DIM), jnp.float32)], axis=1
    )
    zeros_init = jnp.zeros((_N_AGG, _HID), jnp.float32)
    ones_rows = jnp.ones((_CH, _HID), jnp.float32)

    layers = params["layers"]
    h = x_pad
    deg2 = None
    for li, din in enumerate((_IN_DIM, _HID)):
        lp = layers[li]
        xs = _sc_gather_fn()(h, src3)
        msg = _tc_edge(ea_t, xs, lp["W1"], lp["b1"], lp["W2"], lp["b2"], din)
        if li == 0:
            agg2, deg2 = _make_sc_scatter(True)(msg, dst3, zeros_init, ones_rows)
            h = _tc_norm(
                agg2[:, :_N], deg2[:, :_N], lp["bias"], lp["gamma"], lp["beta"]
            )
        else:
            agg2 = _make_sc_scatter(False)(msg, dst3, zeros_init, ones_rows)
            if isinstance(agg2, (list, tuple)):
                agg2 = agg2[0]

    return _tc_final(
        agg2[:, :_N], deg2[:, :_N], lp["bias"], lp["gamma"], lp["beta"],
        node_graph_ids, params["bn_out_gamma"], params["bn_out_beta"]
    )


# speedup vs baseline: 1.1084x; 1.0027x over previous
import functools

import jax
import jax.numpy as jnp
from jax import lax
from jax.experimental import pallas as pl
from jax.experimental.pallas import tpu as pltpu
from jax.experimental.pallas import tpu_sc as plsc

_N = 10000
_E = 160000
_G = 64
_IN_DIM = 11
_HID = 16
_E_DIM = 6
_EDGE_H = 64

try:
    _INFO = plsc.get_sparse_core_info()
    _NC = _INFO.num_cores
    _NS = _INFO.num_subcores
except ValueError:
    _NC, _NS = 2, 16
_NW = _NC * _NS
_CH = 128
_N_CH = 40
_PER_W = _N_CH * _CH
_E_PAD = _NW * _PER_W
_ROWS_S = 640
_N_AGG = _NS * _ROWS_S
_GRP = 20

@functools.lru_cache
def _get_mesh():
    return plsc.VectorSubcoreMesh(
        core_axis_name="c", subcore_axis_name="s", num_cores=_NC, num_subcores=_NS
    )


@functools.lru_cache
def _sc_gather_fn():
    @functools.partial(
        pl.kernel,
        out_type=jax.ShapeDtypeStruct((_E_PAD, _HID), jnp.float32),
        mesh=_get_mesh(),
        scratch_types=[
            pltpu.VMEM((_N_CH, _CH), jnp.int32),
            pltpu.VMEM((_PER_W, _HID), jnp.float32),
            pltpu.SemaphoreType.DMA,
        ],
        compiler_params=pltpu.CompilerParams(
            use_tc_tiling_on_sc=False, skip_device_barrier=True
        ),
    )
    def _sc_gather(table_hbm, idx_hbm, out_hbm, idx_v, rows_v, sem):
        wid = lax.axis_index("s") * _NC + lax.axis_index("c")
        pltpu.sync_copy(idx_hbm.at[wid], idx_v)

        def body(g, carry):
            base = g * _GRP
            descs = [
                pltpu.async_copy(
                    table_hbm.at[idx_v.at[base + t]],
                    rows_v.at[pl.ds((base + t) * _CH, _CH)],
                    sem,
                )
                for t in range(_GRP)
            ]
            for d in descs:
                d.wait()
            return carry

        lax.fori_loop(0, _N_CH // _GRP, body, 0)
        pltpu.sync_copy(rows_v, out_hbm.at[pl.ds(wid * _PER_W, _PER_W)])

    return _sc_gather


@functools.lru_cache
def _make_sc_scatter(with_deg):
    n_out = 2 if with_deg else 1
    out_type = [jax.ShapeDtypeStruct((_NC, _N_AGG, _HID), jnp.float32)] * n_out
    scratch = [
        pltpu.VMEM((_N_CH, _CH), jnp.int32),
        pltpu.VMEM((_PER_W, _HID), jnp.float32),
        pltpu.VMEM((_CH, _HID), jnp.float32),
        pltpu.VMEM_SHARED((_N_AGG, _HID), jnp.float32),
        pltpu.SemaphoreType.DMA,
    ]
    if with_deg:
        scratch.append(pltpu.VMEM_SHARED((_N_AGG, _HID), jnp.float32))

    @functools.partial(
        pl.kernel, out_type=out_type, mesh=_get_mesh(), scratch_types=scratch,
        compiler_params=pltpu.CompilerParams(
            use_tc_tiling_on_sc=False, skip_device_barrier=True
        ),
    )
    def _sc_scatter(msg_hbm, dst_hbm, zeros_hbm, ones_hbm, *rest):
        if with_deg:
            agg_out, deg_out, idx_v, rows_v, ones_v, agg_sh, sem, deg_sh = rest
        else:
            agg_out, idx_v, rows_v, ones_v, agg_sh, sem = rest
        cid = lax.axis_index("c")
        sid = lax.axis_index("s")
        wid = sid * _NC + cid
        r0 = sid * _ROWS_S
        pltpu.sync_copy(zeros_hbm.at[pl.ds(r0, _ROWS_S)], agg_sh.at[pl.ds(r0, _ROWS_S)])
        if with_deg:
            pltpu.sync_copy(zeros_hbm.at[pl.ds(r0, _ROWS_S)], deg_sh.at[pl.ds(r0, _ROWS_S)])
            pltpu.sync_copy(ones_hbm, ones_v)
        pltpu.sync_copy(dst_hbm.at[wid], idx_v)
        pltpu.sync_copy(msg_hbm.at[pl.ds(wid * _PER_W, _PER_W)], rows_v)
        plsc.subcore_barrier()

        def body(j, carry):
            d1 = pltpu.async_copy(
                rows_v.at[pl.ds(j * _CH, _CH)], agg_sh.at[idx_v.at[j]], sem,
                add=True,
            )
            if with_deg:
                d2 = pltpu.async_copy(
                    ones_v, deg_sh.at[idx_v.at[j]], sem, add=True
                )
            d1.wait()
            if with_deg:
                d2.wait()
            return carry

        lax.fori_loop(0, _N_CH, body, 0)
        plsc.subcore_barrier()
        pltpu.sync_copy(agg_sh.at[pl.ds(r0, _ROWS_S)], agg_out.at[cid, pl.ds(r0, _ROWS_S)])
        if with_deg:
            pltpu.sync_copy(deg_sh.at[pl.ds(r0, _ROWS_S)], deg_out.at[cid, pl.ds(r0, _ROWS_S)])

    return _sc_scatter


_EB = 4096


def _tc_edge(ea_t, xs_t, W1, b1, W2, b2, din):
    def body(ea_ref, xs_ref, w1t_ref, b1_ref, w2t_ref, b2_ref, out_ref):
        eh_t = jnp.maximum(
            jnp.dot(w1t_ref[...], ea_ref[...], preferred_element_type=jnp.float32)
            + b1_ref[...],
            0.0,
        )
        p_t = (
            jnp.dot(w2t_ref[...], eh_t, preferred_element_type=jnp.float32)
            + b2_ref[...]
        )
        xs_t_b = xs_ref[...].T
        acc = xs_t_b[0:1, :] * p_t[0:_HID, :]
        for i in range(1, din):
            acc = acc + xs_t_b[i : i + 1, :] * p_t[i * _HID : (i + 1) * _HID, :]
        out_ref[...] = acc.T

    grid = (_E_PAD // _EB,)
    return pl.pallas_call(
        body,
        grid=grid,
        in_specs=[
            pl.BlockSpec((_E_DIM, _EB), lambda i: (0, i)),
            pl.BlockSpec((_EB, _HID), lambda i: (i, 0)),
            pl.BlockSpec((_EDGE_H, _E_DIM), lambda i: (0, 0)),
            pl.BlockSpec((_EDGE_H, 1), lambda i: (0, 0)),
            pl.BlockSpec((din * _HID, _EDGE_H), lambda i: (0, 0)),
            pl.BlockSpec((din * _HID, 1), lambda i: (0, 0)),
        ],
        out_specs=pl.BlockSpec((_EB, _HID), lambda i: (i, 0)),
        out_shape=jax.ShapeDtypeStruct((_E_PAD, _HID), jnp.float32),
    )(ea_t, xs_t, W1.T, b1[:, None], W2.T, b2[:, None])


def _tc_norm(agg2, deg2, bias, gamma, beta):
    def body(a_ref, d_ref, bias_ref, g_ref, be_ref, out_ref):
        a = a_ref[0] + a_ref[1]
        d = d_ref[0] + d_ref[1]
        t = a / jnp.maximum(d, 1.0) + bias_ref[...]
        r = jnp.maximum(t, 0.0)
        m = jnp.mean(r, axis=0, keepdims=True)
        v = jnp.mean((r - m) * (r - m), axis=0, keepdims=True)
        out_ref[...] = (r - m) / jnp.sqrt(v + 1e-5) * g_ref[...] + be_ref[...]

    return pl.pallas_call(
        body,
        out_shape=jax.ShapeDtypeStruct((_N, _HID), jnp.float32),
    )(agg2, deg2, bias[None], gamma[None], beta[None])


def _tc_final(agg2, deg2, bias, gamma, beta, gid, g2, be2):
    def body(a_ref, d_ref, bias_ref, lg_ref, lbe_ref, gid_ref, g_ref, be_ref,
             out_ref, hx_ref):
        a = a_ref[0] + a_ref[1]
        d = d_ref[0] + d_ref[1]
        t = a / jnp.maximum(d, 1.0) + bias_ref[...]
        r = jnp.maximum(t, 0.0)
        lm = jnp.mean(r, axis=0, keepdims=True)
        lv = jnp.mean((r - lm) * (r - lm), axis=0, keepdims=True)
        hv = (r - lm) / jnp.sqrt(lv + 1e-5) * lg_ref[...] + lbe_ref[...]
        gids = gid_ref[...]
        iota_g = lax.broadcasted_iota(jnp.int32, (_N, _G), 1)
        maskf = (gids == iota_g).astype(jnp.float32)
        counts = jnp.sum(maskf, axis=0)[:, None]
        sums = lax.dot_general(
            maskf, hv, (((0,), (0,)), ((), ())),
            preferred_element_type=jnp.float32,
        )
        hn = sums / jnp.maximum(counts, 1.0)
        m = jnp.mean(hn, axis=0, keepdims=True)
        v = jnp.mean((hn - m) * (hn - m), axis=0, keepdims=True)
        hnb = (hn - m) / jnp.sqrt(v + 1e-5) * g_ref[...] + be_ref[...]

        def body_g(g, carry):
            mk = gids == g
            mx = jnp.max(jnp.where(mk, hv, -jnp.inf), axis=0)
            hx_ref[pl.ds(g, 1), :] = mx[None, :]
            return carry

        lax.fori_loop(0, _G, body_g, 0)
        out_ref[:, 0:_HID] = hnb
        out_ref[:, _HID : 2 * _HID] = hx_ref[...]

    return pl.pallas_call(
        body,
        out_shape=jax.ShapeDtypeStruct((_G, 2 * _HID), jnp.float32),
        scratch_shapes=[pltpu.VMEM((_G, _HID), jnp.float32)],
    )(agg2, deg2, bias[None], gamma[None], beta[None], gid[:, None],
      g2[None], be2[None])


def kernel(x, edge_index, edge_attr, node_graph_ids, params):
    src = edge_index[0]
    dst = edge_index[1]
    pad_e = _E_PAD - _E
    src3 = jnp.concatenate([src, jnp.zeros((pad_e,), jnp.int32)]).reshape(
        _NW, _N_CH, _CH
    )
    dst3 = jnp.concatenate([dst, jnp.full((pad_e,), _N, jnp.int32)]).reshape(
        _NW, _N_CH, _CH
    )
    ea_t = jnp.concatenate(
        [edge_attr, jnp.zeros((pad_e, _E_DIM), jnp.float32)], axis=0
    ).T
    x_pad = jnp.concatenate(
        [x, jnp.zeros((_N, _HID - _IN_DIM), jnp.float32)], axis=1
    )
    zeros_init = jnp.zeros((_N_AGG, _HID), jnp.float32)
    ones_rows = jnp.ones((_CH, _HID), jnp.float32)

    layers = params["layers"]
    h = x_pad
    deg2 = None
    for li, din in enumerate((_IN_DIM, _HID)):
        lp = layers[li]
        xs = _sc_gather_fn()(h, src3)
        msg = _tc_edge(ea_t, xs, lp["W1"], lp["b1"], lp["W2"], lp["b2"], din)
        if li == 0:
            agg2, deg2 = _make_sc_scatter(True)(msg, dst3, zeros_init, ones_rows)
            h = _tc_norm(
                agg2[:, :_N], deg2[:, :_N], lp["bias"], lp["gamma"], lp["beta"]
            )
        else:
            agg2 = _make_sc_scatter(False)(msg, dst3, zeros_init, ones_rows)
            if isinstance(agg2, (list, tuple)):
                agg2 = agg2[0]

    return _tc_final(
        agg2[:, :_N], deg2[:, :_N], lp["bias"], lp["gamma"], lp["beta"],
        node_graph_ids, params["bn_out_gamma"], params["bn_out_beta"]
    )
